# Initial kernel scaffold; baseline (speedup 1.0000x reference)
import jax
import jax.numpy as jnp
from jax.experimental import pallas as pl


def kernel(index, pe):
    raise NotImplementedError("write your pallas kernel here")


# speedup vs baseline: 6.6508x; 6.6508x over previous
import functools

import jax
import jax.numpy as jnp
from jax import lax
from jax.experimental import pallas as pl
from jax.experimental.pallas import tpu as pltpu
from jax.experimental.pallas import tpu_sc as plsc

D_TABLE = 128
BLK = 128
NUM_CORES = 2
NUM_SUBCORES = 16
NW = NUM_CORES * NUM_SUBCORES
K = 2


@functools.partial(jax.jit, static_argnames=("n_blocks",))
def _sc_gather(idx2d, pe, n_blocks):
    blocks_per_w = n_blocks // NW
    G = blocks_per_w // K
    mesh = plsc.VectorSubcoreMesh(core_axis_name="c", subcore_axis_name="s")

    @functools.partial(
        pl.kernel,
        mesh=mesh,
        out_type=jax.ShapeDtypeStruct((n_blocks, BLK, D_TABLE), jnp.float32),
        scratch_types=[
            pltpu.VMEM((K, BLK), jnp.int32),
            pltpu.VMEM((K, BLK, D_TABLE), jnp.float32),
            pltpu.SemaphoreType.DMA,
        ],
    )
    def k(idx_hbm, pe_hbm, out_hbm, idx_v, rows_v, gsem):
        wid = lax.axis_index("s") * NUM_CORES + lax.axis_index("c")
        base = wid * blocks_per_w

        def body(g, carry):
            row0 = base + g * K
            pltpu.sync_copy(idx_hbm.at[pl.ds(row0, K)], idx_v)
            gathers = [
                pltpu.async_copy(pe_hbm.at[idx_v.at[j]], rows_v.at[j], gsem)
                for j in range(K)
            ]
            for g_ in gathers:
                g_.wait()
            pltpu.sync_copy(rows_v, out_hbm.at[pl.ds(row0, K)])
            return carry

        lax.fori_loop(0, G, body, 0)

    return k(idx2d, pe)


def kernel(index, pe):
    b, t = index.shape
    n = b * t
    n_blocks = n // BLK
    idx2d = index.reshape(n_blocks, BLK)
    out = _sc_gather(idx2d, pe, n_blocks)
    return out.reshape(b, t, D_TABLE)


# speedup vs baseline: 6.8264x; 1.0264x over previous
import functools

import jax
import jax.numpy as jnp
from jax import lax
from jax.experimental import pallas as pl
from jax.experimental.pallas import tpu as pltpu
from jax.experimental.pallas import tpu_sc as plsc

D_TABLE = 128
BLK = 128
NUM_CORES = 2
NUM_SUBCORES = 16
NW = NUM_CORES * NUM_SUBCORES
K = 2


@functools.partial(jax.jit, static_argnames=("n_blocks",))
def _sc_gather(idx2d, pe, n_blocks):
    blocks_per_w = n_blocks // NW
    G = blocks_per_w // K
    mesh = plsc.VectorSubcoreMesh(core_axis_name="c", subcore_axis_name="s")

    @functools.partial(
        pl.kernel,
        mesh=mesh,
        out_type=jax.ShapeDtypeStruct((n_blocks, BLK, D_TABLE), jnp.float32),
        scratch_types=[
            pltpu.VMEM((K, BLK), jnp.int32),
            pltpu.VMEM((K, BLK), jnp.int32),
            pltpu.VMEM((K, BLK, D_TABLE), jnp.float32),
            pltpu.VMEM((K, BLK, D_TABLE), jnp.float32),
            pltpu.SemaphoreType.DMA,
            pltpu.SemaphoreType.DMA,
            pltpu.SemaphoreType.DMA,
        ],
    )
    def k(idx_hbm, pe_hbm, out_hbm, ib0, ib1, rows0, rows1, gsem, ssem, isem):
        wid = lax.axis_index("s") * NUM_CORES + lax.axis_index("c")
        base = wid * blocks_per_w
        ibufs = (ib0, ib1)
        rbufs = (rows0, rows1)

        def fire_gathers(ib, rb):
            return [
                pltpu.async_copy(pe_hbm.at[ib.at[j]], rb.at[j], gsem)
                for j in range(K)
            ]

        def step(g, b, first):
            ib, ibn = ibufs[b], ibufs[1 - b]
            rb, rbn = rbufs[b], rbufs[1 - b]
            row0 = base + g * K
            gd = fire_gathers(ib, rb)
            nxt = base + jnp.minimum(g + 1, G - 1) * K
            idesc = pltpu.async_copy(idx_hbm.at[pl.ds(nxt, K)], ibn, isem)
            if not first:
                pltpu.make_async_copy(rbn, out_hbm.at[pl.ds(row0, K)], ssem).wait()
            for d in gd:
                d.wait()
            pltpu.async_copy(rb, out_hbm.at[pl.ds(row0, K)], ssem)
            idesc.wait()

        pltpu.sync_copy(idx_hbm.at[pl.ds(base, K)], ib0)
        step(0, 0, True)
        step(1, 1, False)

        def body(t, carry):
            g = 2 + 2 * t
            step(g, 0, False)
            step(g + 1, 1, False)
            return carry

        lax.fori_loop(0, (G - 2) // 2, body, 0)

        pltpu.make_async_copy(rbufs[(G - 1) % 2], out_hbm.at[pl.ds(base, K)], ssem).wait()

    return k(idx2d, pe)


def kernel(index, pe):
    b, t = index.shape
    n = b * t
    n_blocks = n // BLK
    idx2d = index.reshape(n_blocks, BLK)
    out = _sc_gather(idx2d, pe, n_blocks)
    return out.reshape(b, t, D_TABLE)


# speedup vs baseline: 19.0660x; 2.7930x over previous
import functools

import jax
import jax.numpy as jnp
from jax import lax
from jax.experimental import pallas as pl
from jax.experimental.pallas import tpu as pltpu
from jax.experimental.pallas import tpu_sc as plsc

D_TABLE = 128
BLK = 128
NUM_CORES = 2
NUM_SUBCORES = 16
NW = NUM_CORES * NUM_SUBCORES
K = 2


@functools.partial(jax.jit, static_argnames=("n_blocks",))
def _sc_gather(idx2d, pe, n_blocks):
    blocks_per_w = n_blocks // NW
    G = blocks_per_w // K
    mesh = plsc.VectorSubcoreMesh(core_axis_name="c", subcore_axis_name="s")

    @functools.partial(
        pl.kernel,
        mesh=mesh,
        out_type=jax.ShapeDtypeStruct((n_blocks, BLK, D_TABLE), jnp.float32),
        scratch_types=[
            pltpu.VMEM((K, BLK), jnp.int32),
            pltpu.VMEM((K, BLK), jnp.int32),
            pltpu.VMEM((K, BLK, D_TABLE), jnp.float32),
            pltpu.VMEM((K, BLK, D_TABLE), jnp.float32),
            pltpu.VMEM_SHARED((1024, D_TABLE), jnp.float32),
            pltpu.SemaphoreType.DMA,
            pltpu.SemaphoreType.DMA,
            pltpu.SemaphoreType.DMA,
        ],
    )
    def k(idx_hbm, pe_hbm, out_hbm, ib0, ib1, rows0, rows1, pe_sh, gsem, ssem, isem):
        wid = lax.axis_index("s") * NUM_CORES + lax.axis_index("c")
        base = wid * blocks_per_w
        ibufs = (ib0, ib1)
        rbufs = (rows0, rows1)

        @pl.when(lax.axis_index("s") == 0)
        def _stage():
            pltpu.sync_copy(pe_hbm, pe_sh)

        plsc.subcore_barrier()

        def fire_gathers(ib, rb):
            return [
                pltpu.async_copy(pe_sh.at[ib.at[j]], rb.at[j], gsem)
                for j in range(K)
            ]

        def step(g, b, first):
            ib, ibn = ibufs[b], ibufs[1 - b]
            rb, rbn = rbufs[b], rbufs[1 - b]
            row0 = base + g * K
            gd = fire_gathers(ib, rb)
            nxt = base + jnp.minimum(g + 1, G - 1) * K
            idesc = pltpu.async_copy(idx_hbm.at[pl.ds(nxt, K)], ibn, isem)
            if not first:
                pltpu.make_async_copy(rbn, out_hbm.at[pl.ds(row0, K)], ssem).wait()
            for d in gd:
                d.wait()
            pltpu.async_copy(rb, out_hbm.at[pl.ds(row0, K)], ssem)
            idesc.wait()

        pltpu.sync_copy(idx_hbm.at[pl.ds(base, K)], ib0)
        step(0, 0, True)
        step(1, 1, False)

        def body(t, carry):
            g = 2 + 2 * t
            step(g, 0, False)
            step(g + 1, 1, False)
            return carry

        lax.fori_loop(0, (G - 2) // 2, body, 0)

        pltpu.make_async_copy(rbufs[(G - 1) % 2], out_hbm.at[pl.ds(base, K)], ssem).wait()

    return k(idx2d, pe)


def kernel(index, pe):
    b, t = index.shape
    n = b * t
    n_blocks = n // BLK
    idx2d = index.reshape(n_blocks, BLK)
    out = _sc_gather(idx2d, pe, n_blocks)
    return out.reshape(b, t, D_TABLE)
